# baseline (speedup 1.0000x reference)
import functools
from math import sqrt

import jax
import jax.numpy as jnp
from jax.experimental import pallas as pl
from jax.experimental.pallas import tpu as pltpu

N = 10000
E = 320000
F = 100000
ENC = 128
DEC = 128
CH = 8
SH = 8
HD = 16
NEG_SLOPE = 0.1
HFFP = 384

ROW_BLK = 1000


def _swish(x):
    return x * jax.nn.sigmoid(x)


def _rms_norm(x, w):
    return x / jnp.sqrt(jnp.mean(x * x, axis=-1, keepdims=True) + 1e-6) * w


def _pre1_body(fb_ref, root_ref, wa_ref, wc_ref, src_t_ref, xp_t_ref):
    fb = fb_ref[...]
    root = root_ref[...]
    ctxp = jnp.dot(fb, wc_ref[...], preferred_element_type=jnp.float32)
    cp = jnp.dot(fb, wa_ref[...][:DEC], preferred_element_type=jnp.float32)
    xp = jnp.dot(root, wa_ref[...][DEC:], preferred_element_type=jnp.float32)
    zeros8 = jnp.zeros((fb.shape[0], 8), jnp.float32)
    src_t_ref[...] = jnp.concatenate([ctxp, cp, zeros8], axis=1)
    xp_t_ref[...] = jnp.concatenate([xp, zeros8], axis=1)


def _pre1(fb, root, w_attn, w_ctx2x):
    grid = (N // ROW_BLK,)
    return pl.pallas_call(
        _pre1_body,
        grid=grid,
        in_specs=[
            pl.BlockSpec((ROW_BLK, DEC), lambda i: (i, 0)),
            pl.BlockSpec((ROW_BLK, ENC), lambda i: (i, 0)),
            pl.BlockSpec((DEC + ENC, CH), lambda i: (0, 0)),
            pl.BlockSpec((DEC, ENC), lambda i: (0, 0)),
        ],
        out_specs=[
            pl.BlockSpec((ROW_BLK, 144), lambda i: (i, 0)),
            pl.BlockSpec((ROW_BLK, 16), lambda i: (0, 0) if False else (i, 0)),
        ],
        out_shape=[
            jax.ShapeDtypeStruct((N, 144), jnp.float32),
            jax.ShapeDtypeStruct((N, 16), jnp.float32),
        ],
    )(fb, root, w_attn, w_ctx2x)


def _mid_body(acc_ref, root_ref, wx2g_ref, wrms_ref, wq_ref, wkv_ref, p_ref,
              root1_ref, q_ref, kv_ref):
    acc = acc_ref[0] + acc_ref[1]
    root = root_ref[...]
    p = p_ref[...]
    s = acc[:, 128:136]
    unm = acc[:, :128]
    inv = 1.0 / (s + 1e-16)
    mess = unm * jnp.dot(inv, p, preferred_element_type=jnp.float32)
    gates = jnp.dot(
        jnp.dot(root, wx2g_ref[...], preferred_element_type=jnp.float32),
        p, preferred_element_type=jnp.float32)
    has_in = jnp.broadcast_to(s[:, :1] > 0, gates.shape)
    gates = jnp.where(has_in, gates, 1.0)
    out = gates * root + (1.0 - gates) * mess
    root1 = _rms_norm(root + out, wrms_ref[...][0])
    root1_ref[...] = root1
    q_ref[...] = jnp.dot(root1, wq_ref[...], preferred_element_type=jnp.float32)
    kv_ref[...] = jnp.dot(root1, wkv_ref[...], preferred_element_type=jnp.float32)


def _mid(acc, root, w_x2g, rms_node_w, wq, wkv, p):
    grid = (N // ROW_BLK,)
    return pl.pallas_call(
        _mid_body,
        grid=grid,
        in_specs=[
            pl.BlockSpec((2, ROW_BLK, 144), lambda i: (0, i, 0)),
            pl.BlockSpec((ROW_BLK, ENC), lambda i: (i, 0)),
            pl.BlockSpec((ENC, CH), lambda i: (0, 0)),
            pl.BlockSpec((1, ENC), lambda i: (0, 0)),
            pl.BlockSpec((ENC, ENC), lambda i: (0, 0)),
            pl.BlockSpec((ENC, 2 * ENC), lambda i: (0, 0)),
            pl.BlockSpec((CH, ENC), lambda i: (0, 0)),
        ],
        out_specs=[
            pl.BlockSpec((ROW_BLK, ENC), lambda i: (i, 0)),
            pl.BlockSpec((ROW_BLK, ENC), lambda i: (i, 0)),
            pl.BlockSpec((ROW_BLK, 2 * ENC), lambda i: (i, 0)),
        ],
        out_shape=[
            jax.ShapeDtypeStruct((N, ENC), jnp.float32),
            jax.ShapeDtypeStruct((N, ENC), jnp.float32),
            jax.ShapeDtypeStruct((N, 2 * ENC), jnp.float32),
        ],
    )(acc, root, w_x2g, rms_node_w.reshape(1, ENC), wq, wkv, p)


def _post_body(acc_ref, root1_ref, wr_ref, wg_ref, wu_ref, wd_ref, wf_ref,
               p_ref, root3_ref):
    acc = acc_ref[0] + acc_ref[1]
    root1 = root1_ref[...]
    s = acc[:, 128:136]
    inv = 1.0 / (s + 1e-16)
    mha = acc[:, :128] * jnp.dot(inv, p_ref[...], preferred_element_type=jnp.float32)
    root2 = _rms_norm(root1 + mha, wr_ref[...][0])
    gate = jnp.dot(root2, wg_ref[...], preferred_element_type=jnp.float32)
    up = jnp.dot(root2, wu_ref[...], preferred_element_type=jnp.float32)
    ffn = jnp.dot(_swish(gate) * up, wd_ref[...], preferred_element_type=jnp.float32)
    root3_ref[...] = _rms_norm(root2 + ffn, wf_ref[...][0])


def _post(acc, root1, rms_root_w, wg, wu, wd, rms_ffn_w, p):
    grid = (N // ROW_BLK,)
    return pl.pallas_call(
        _post_body,
        grid=grid,
        in_specs=[
            pl.BlockSpec((2, ROW_BLK, 144), lambda i: (0, i, 0)),
            pl.BlockSpec((ROW_BLK, ENC), lambda i: (i, 0)),
            pl.BlockSpec((1, ENC), lambda i: (0, 0)),
            pl.BlockSpec((ENC, HFFP), lambda i: (0, 0)),
            pl.BlockSpec((ENC, HFFP), lambda i: (0, 0)),
            pl.BlockSpec((HFFP, ENC), lambda i: (0, 0)),
            pl.BlockSpec((1, ENC), lambda i: (0, 0)),
            pl.BlockSpec((CH, ENC), lambda i: (0, 0)),
        ],
        out_specs=pl.BlockSpec((ROW_BLK, ENC), lambda i: (i, 0)),
        out_shape=jax.ShapeDtypeStruct((N, ENC), jnp.float32),
    )(acc, root1, rms_root_w.reshape(1, ENC), wg, wu, wd,
      rms_ffn_w.reshape(1, ENC), p)


FR_BLK = 1000


def _fringe_body(fr_ref, gr_ref, wfg_ref, bfg_ref, wrf_ref, brf_ref, out_ref):
    fr = fr_ref[...]
    fg = _swish(jnp.dot(fr, wfg_ref[...], preferred_element_type=jnp.float32)
                + bfg_ref[...][0])
    r2f = jnp.dot(gr_ref[...], wrf_ref[...], preferred_element_type=jnp.float32) \
        + brf_ref[...][0]
    out_ref[...] = r2f * fg


def _fringe(fr, gathered, w_fgate, b_fgate, w_r2f, b_r2f):
    grid = (F // FR_BLK,)
    return pl.pallas_call(
        _fringe_body,
        grid=grid,
        in_specs=[
            pl.BlockSpec((FR_BLK, DEC), lambda i: (i, 0)),
            pl.BlockSpec((FR_BLK, ENC), lambda i: (i, 0)),
            pl.BlockSpec((DEC, DEC), lambda i: (0, 0)),
            pl.BlockSpec((1, DEC), lambda i: (0, 0)),
            pl.BlockSpec((ENC, DEC), lambda i: (0, 0)),
            pl.BlockSpec((1, DEC), lambda i: (0, 0)),
        ],
        out_specs=pl.BlockSpec((FR_BLK, DEC), lambda i: (i, 0)),
        out_shape=jax.ShapeDtypeStruct((F, DEC), jnp.float32),
    )(fr, gathered, w_fgate, b_fgate.reshape(1, DEC), w_r2f,
      b_r2f.reshape(1, DEC))


def _edge1_jnp(src, dst, src_table, xp_table):
    z = src_table[src, 128:136] + xp_table[dst, :8]
    a = jnp.maximum(z, NEG_SLOPE * z)
    w = jnp.exp(a)
    rows = jnp.concatenate([
        jnp.repeat(w, HD, axis=1) * src_table[src, :128],
        w, jnp.ones((E, 8), jnp.float32)], axis=1)
    acc = jnp.zeros((N, 144), jnp.float32).at[dst].add(rows)
    return jnp.stack([acc, jnp.zeros_like(acc)])


def _edge2_jnp(s2, d2, q_table, kv_table, attr):
    qg = q_table[d2].reshape(E, SH, HD)
    k = kv_table[s2, :128].reshape(E, SH, HD)
    v = kv_table[s2, 128:].reshape(E, SH, HD)
    atn = (qg * k * attr[:, None, :]).sum(-1) / 4.0
    w = jnp.exp(atn)
    rows = jnp.concatenate([
        (w[:, :, None] * v).reshape(E, 128), w,
        jnp.ones((E, 8), jnp.float32)], axis=1)
    acc = jnp.zeros((N, 144), jnp.float32).at[d2].add(rows)
    return jnp.stack([acc, jnp.zeros_like(acc)])


def _fgather_jnp(root3, idx):
    return root3[idx]


def kernel(root_features, feedback_features, feedback_index, fringe_features,
           root_to_fringe_index, root_edge_index, root_edge_attr, W_attn,
           W_ctx2x, W_x2g, W_qkv, W_gate, W_up, W_down, W_fgate, b_fgate,
           W_r2f, b_r2f, rms_node_w, rms_root_w, rms_ffn_w):
    p = jnp.kron(jnp.eye(CH, dtype=jnp.float32), jnp.ones((1, HD), jnp.float32))
    wqkv4 = W_qkv.reshape(ENC, SH, HD, 3)
    wq = wqkv4[..., 0].reshape(ENC, ENC)
    wkv = jnp.concatenate(
        [wqkv4[..., 1].reshape(ENC, ENC), wqkv4[..., 2].reshape(ENC, ENC)],
        axis=1)
    hff = W_gate.shape[1]
    wg = jnp.pad(W_gate, ((0, 0), (0, HFFP - hff)))
    wu = jnp.pad(W_up, ((0, 0), (0, HFFP - hff)))
    wd = jnp.pad(W_down, ((0, HFFP - hff), (0, 0)))

    src = feedback_index[0].astype(jnp.int32)
    dst = feedback_index[1].astype(jnp.int32)
    s2 = root_edge_index[0].astype(jnp.int32)
    d2 = root_edge_index[1].astype(jnp.int32)

    src_table, xp_table = _pre1(feedback_features, root_features, W_attn,
                                W_ctx2x)
    acc1 = _edge1_jnp(src, dst, src_table, xp_table)
    root1, q_table, kv_table = _mid(acc1, root_features, W_x2g, rms_node_w,
                                    wq, wkv, p)
    acc2 = _edge2_jnp(s2, d2, q_table, kv_table, root_edge_attr)
    root3 = _post(acc2, root1, rms_root_w, wg, wu, wd, rms_ffn_w, p)
    gathered = _fgather_jnp(root3, root_to_fringe_index.astype(jnp.int32))
    fringe_out = _fringe(fringe_features, gathered, W_fgate, b_fgate, W_r2f,
                         b_r2f)
    return (root3, fringe_out)


# speedup vs baseline: 116.1090x; 116.1090x over previous
import functools
from math import sqrt

import jax
import jax.numpy as jnp
from jax import lax
from jax.experimental import pallas as pl
from jax.experimental.pallas import tpu as pltpu
from jax.experimental.pallas import tpu_sc as plsc

N = 10000
E = 320000
F = 100000
ENC = 128
DEC = 128
CH = 8
SH = 8
HD = 16
NEG_SLOPE = 0.1
HFFP = 384

ROW_BLK = 1000


def _swish(x):
    return x * jax.nn.sigmoid(x)


def _rms_norm(x, w):
    return x / jnp.sqrt(jnp.mean(x * x, axis=-1, keepdims=True) + 1e-6) * w


def _pre1_body(fb_ref, root_ref, wa_ref, wc_ref, src_t_ref, xp_t_ref):
    fb = fb_ref[...]
    root = root_ref[...]
    ctxp = jnp.dot(fb, wc_ref[...], preferred_element_type=jnp.float32)
    cp = jnp.dot(fb, wa_ref[...][:DEC], preferred_element_type=jnp.float32)
    xp = jnp.dot(root, wa_ref[...][DEC:], preferred_element_type=jnp.float32)
    zeros8 = jnp.zeros((fb.shape[0], 8), jnp.float32)
    src_t_ref[...] = jnp.concatenate([ctxp, cp, zeros8], axis=1)
    xp_t_ref[...] = jnp.concatenate([xp, zeros8], axis=1)


def _pre1(fb, root, w_attn, w_ctx2x):
    grid = (N // ROW_BLK,)
    return pl.pallas_call(
        _pre1_body,
        grid=grid,
        in_specs=[
            pl.BlockSpec((ROW_BLK, DEC), lambda i: (i, 0)),
            pl.BlockSpec((ROW_BLK, ENC), lambda i: (i, 0)),
            pl.BlockSpec((DEC + ENC, CH), lambda i: (0, 0)),
            pl.BlockSpec((DEC, ENC), lambda i: (0, 0)),
        ],
        out_specs=[
            pl.BlockSpec((ROW_BLK, 144), lambda i: (i, 0)),
            pl.BlockSpec((ROW_BLK, 16), lambda i: (0, 0) if False else (i, 0)),
        ],
        out_shape=[
            jax.ShapeDtypeStruct((N, 144), jnp.float32),
            jax.ShapeDtypeStruct((N, 16), jnp.float32),
        ],
    )(fb, root, w_attn, w_ctx2x)


def _mid_body(acc_ref, root_ref, wx2g_ref, wrms_ref, wq_ref, wkv_ref, p_ref,
              root1_ref, q_ref, kv_ref):
    acc = acc_ref[0] + acc_ref[1]
    root = root_ref[...]
    p = p_ref[...]
    s = acc[:, 128:136]
    unm = acc[:, :128]
    inv = 1.0 / (s + 1e-16)
    mess = unm * jnp.dot(inv, p, preferred_element_type=jnp.float32)
    gates = jnp.dot(
        jnp.dot(root, wx2g_ref[...], preferred_element_type=jnp.float32),
        p, preferred_element_type=jnp.float32)
    has_in = jnp.broadcast_to(s[:, :1] > 0, gates.shape)
    gates = jnp.where(has_in, gates, 1.0)
    out = gates * root + (1.0 - gates) * mess
    root1 = _rms_norm(root + out, wrms_ref[...][0])
    root1_ref[...] = root1
    q_ref[...] = jnp.dot(root1, wq_ref[...], preferred_element_type=jnp.float32)
    kv_ref[...] = jnp.dot(root1, wkv_ref[...], preferred_element_type=jnp.float32)


def _mid(acc, root, w_x2g, rms_node_w, wq, wkv, p):
    grid = (N // ROW_BLK,)
    return pl.pallas_call(
        _mid_body,
        grid=grid,
        in_specs=[
            pl.BlockSpec((2, ROW_BLK, 144), lambda i: (0, i, 0)),
            pl.BlockSpec((ROW_BLK, ENC), lambda i: (i, 0)),
            pl.BlockSpec((ENC, CH), lambda i: (0, 0)),
            pl.BlockSpec((1, ENC), lambda i: (0, 0)),
            pl.BlockSpec((ENC, ENC), lambda i: (0, 0)),
            pl.BlockSpec((ENC, 2 * ENC), lambda i: (0, 0)),
            pl.BlockSpec((CH, ENC), lambda i: (0, 0)),
        ],
        out_specs=[
            pl.BlockSpec((ROW_BLK, ENC), lambda i: (i, 0)),
            pl.BlockSpec((ROW_BLK, ENC), lambda i: (i, 0)),
            pl.BlockSpec((ROW_BLK, 2 * ENC), lambda i: (i, 0)),
        ],
        out_shape=[
            jax.ShapeDtypeStruct((N, ENC), jnp.float32),
            jax.ShapeDtypeStruct((N, ENC), jnp.float32),
            jax.ShapeDtypeStruct((N, 2 * ENC), jnp.float32),
        ],
    )(acc, root, w_x2g, rms_node_w.reshape(1, ENC), wq, wkv, p)


def _post_body(acc_ref, root1_ref, wr_ref, wg_ref, wu_ref, wd_ref, wf_ref,
               p_ref, root3_ref):
    acc = acc_ref[0] + acc_ref[1]
    root1 = root1_ref[...]
    s = acc[:, 128:136]
    inv = 1.0 / (s + 1e-16)
    mha = acc[:, :128] * jnp.dot(inv, p_ref[...], preferred_element_type=jnp.float32)
    root2 = _rms_norm(root1 + mha, wr_ref[...][0])
    gate = jnp.dot(root2, wg_ref[...], preferred_element_type=jnp.float32)
    up = jnp.dot(root2, wu_ref[...], preferred_element_type=jnp.float32)
    ffn = jnp.dot(_swish(gate) * up, wd_ref[...], preferred_element_type=jnp.float32)
    root3_ref[...] = _rms_norm(root2 + ffn, wf_ref[...][0])


def _post(acc, root1, rms_root_w, wg, wu, wd, rms_ffn_w, p):
    grid = (N // ROW_BLK,)
    return pl.pallas_call(
        _post_body,
        grid=grid,
        in_specs=[
            pl.BlockSpec((2, ROW_BLK, 144), lambda i: (0, i, 0)),
            pl.BlockSpec((ROW_BLK, ENC), lambda i: (i, 0)),
            pl.BlockSpec((1, ENC), lambda i: (0, 0)),
            pl.BlockSpec((ENC, HFFP), lambda i: (0, 0)),
            pl.BlockSpec((ENC, HFFP), lambda i: (0, 0)),
            pl.BlockSpec((HFFP, ENC), lambda i: (0, 0)),
            pl.BlockSpec((1, ENC), lambda i: (0, 0)),
            pl.BlockSpec((CH, ENC), lambda i: (0, 0)),
        ],
        out_specs=pl.BlockSpec((ROW_BLK, ENC), lambda i: (i, 0)),
        out_shape=jax.ShapeDtypeStruct((N, ENC), jnp.float32),
    )(acc, root1, rms_root_w.reshape(1, ENC), wg, wu, wd,
      rms_ffn_w.reshape(1, ENC), p)


FR_BLK = 1000


def _fringe_body(fr_ref, gr_ref, wfg_ref, bfg_ref, wrf_ref, brf_ref, out_ref):
    fr = fr_ref[...]
    fg = _swish(jnp.dot(fr, wfg_ref[...], preferred_element_type=jnp.float32)
                + bfg_ref[...][0])
    r2f = jnp.dot(gr_ref[...], wrf_ref[...], preferred_element_type=jnp.float32) \
        + brf_ref[...][0]
    out_ref[...] = r2f * fg


def _fringe(fr, gathered, w_fgate, b_fgate, w_r2f, b_r2f):
    grid = (F // FR_BLK,)
    return pl.pallas_call(
        _fringe_body,
        grid=grid,
        in_specs=[
            pl.BlockSpec((FR_BLK, DEC), lambda i: (i, 0)),
            pl.BlockSpec((FR_BLK, ENC), lambda i: (i, 0)),
            pl.BlockSpec((DEC, DEC), lambda i: (0, 0)),
            pl.BlockSpec((1, DEC), lambda i: (0, 0)),
            pl.BlockSpec((ENC, DEC), lambda i: (0, 0)),
            pl.BlockSpec((1, DEC), lambda i: (0, 0)),
        ],
        out_specs=pl.BlockSpec((FR_BLK, DEC), lambda i: (i, 0)),
        out_shape=jax.ShapeDtypeStruct((F, DEC), jnp.float32),
    )(fr, gathered, w_fgate, b_fgate.reshape(1, DEC), w_r2f,
      b_r2f.reshape(1, DEC))


NC, NS = 2, 16
NP = 10240
NW = NC * NS
EC = 64
NCHUNK = E // EC
KMAX = (NCHUNK + NW - 1) // NW
TR = NP // NS

_MESH = dict(core_axis_name="c", subcore_axis_name="s", num_cores=NC,
             num_subcores=NS)
_LANE_IOTA = None


def _splat16(v, i):
    idx = jnp.full((16, 1), i, jnp.int32)
    return lax.gather(
        v, idx,
        lax.GatherDimensionNumbers(offset_dims=(), collapsed_slice_dims=(0,),
                                   start_index_map=(0,)),
        (1,), mode=lax.GatherScatterMode.PROMISE_IN_BOUNDS)


def _acc_writeback(acc_sh, out_hbm, cid, sid):
    plsc.subcore_barrier()
    base = sid * TR
    pltpu.sync_copy(acc_sh.at[pl.ds(base, TR)],
                    out_hbm.at[pl.ds(cid * NP + base, TR)])


def _acc_zero(zeros_hbm, acc_sh, sid):
    base = sid * TR
    pltpu.sync_copy(zeros_hbm.at[pl.ds(base, TR)], acc_sh.at[pl.ds(base, TR)])
    plsc.subcore_barrier()


def _edge1_sc(src, dst, src_table, xp_table, zeros):
    mesh = plsc.VectorSubcoreMesh(**_MESH)

    @functools.partial(
        pl.kernel,
        out_type=jax.ShapeDtypeStruct((2 * NP, 144), jnp.float32),
        mesh=mesh,
        compiler_params=pltpu.CompilerParams(use_tc_tiling_on_sc=False, needs_layout_passes=False),
        scratch_types=[
            pltpu.VMEM_SHARED((NP, 144), jnp.float32),
            pltpu.VMEM((EC,), jnp.int32),
            pltpu.VMEM((EC,), jnp.int32),
            pltpu.VMEM((EC, 144), jnp.float32),
            pltpu.VMEM((EC, 16), jnp.float32),
        ],
    )
    def k(src_hbm, dst_hbm, st_hbm, xp_hbm, z_hbm, out_hbm,
          acc_sh, sidx, didx, rows, xpr):
        cid = lax.axis_index("c")
        sid = lax.axis_index("s")
        wid = sid * NC + cid
        _acc_zero(z_hbm, acc_sh, sid)

        def chunk(ci, _):
            eb = (wid + ci * NW) * EC
            pltpu.sync_copy(src_hbm.at[pl.ds(eb, EC)], sidx)
            pltpu.sync_copy(dst_hbm.at[pl.ds(eb, EC)], didx)
            pltpu.sync_copy(st_hbm.at[sidx], rows)
            pltpu.sync_copy(xp_hbm.at[didx], xpr)

            def ebody(e, _):
                cp = rows[e, pl.ds(128, 16)]
                z = cp + xpr[e, :]
                a = jnp.maximum(z, NEG_SLOPE * z)
                w = jnp.exp(a)
                rows[e, pl.ds(128, 16)] = w
                for h in range(CH):
                    wh = _splat16(w, h)
                    rows[e, pl.ds(h * HD, HD)] = wh * rows[e, pl.ds(h * HD, HD)]
                return 0

            lax.fori_loop(0, EC, ebody, 0)
            pltpu.sync_copy(rows, acc_sh.at[didx], add=True)
            return 0

        def guarded(ci, _):
            @pl.when(wid + ci * NW < NCHUNK)
            def _():
                chunk(ci, 0)
            return 0

        lax.fori_loop(0, KMAX, guarded, 0)
        _acc_writeback(acc_sh, out_hbm, cid, sid)

    return k(src, dst, src_table, xp_table, zeros).reshape(2, NP, 144)


def _edge2_sc(s2, d2, q_table, kv_table, attr, zeros):
    mesh = plsc.VectorSubcoreMesh(**_MESH)

    @functools.partial(
        pl.kernel,
        out_type=jax.ShapeDtypeStruct((2 * NP, 144), jnp.float32),
        mesh=mesh,
        compiler_params=pltpu.CompilerParams(use_tc_tiling_on_sc=False, needs_layout_passes=False),
        scratch_types=[
            pltpu.VMEM_SHARED((NP, 144), jnp.float32),
            pltpu.VMEM((EC,), jnp.int32),
            pltpu.VMEM((EC,), jnp.int32),
            pltpu.VMEM((EC, 128), jnp.float32),
            pltpu.VMEM((EC, 256), jnp.float32),
            pltpu.VMEM((EC, 16), jnp.float32),
            pltpu.VMEM((EC, 144), jnp.float32),
        ],
    )
    def k(s2_hbm, d2_hbm, q_hbm, kv_hbm, at_hbm, z_hbm, out_hbm,
          acc_sh, sidx, didx, qrows, kvrows, arows, orows):
        cid = lax.axis_index("c")
        sid = lax.axis_index("s")
        wid = sid * NC + cid
        _acc_zero(z_hbm, acc_sh, sid)
        lane_iota = lax.iota(jnp.int32, 16)

        def chunk(ci, _):
            eb = (wid + ci * NW) * EC
            pltpu.sync_copy(s2_hbm.at[pl.ds(eb, EC)], sidx)
            pltpu.sync_copy(d2_hbm.at[pl.ds(eb, EC)], didx)
            pltpu.sync_copy(q_hbm.at[didx], qrows)
            pltpu.sync_copy(kv_hbm.at[sidx], kvrows)
            pltpu.sync_copy(at_hbm.at[pl.ds(eb, EC)], arows)

            def ebody(e, _):
                attr_v = arows[e, :]
                atn = jnp.zeros((16,), jnp.float32)
                for h in range(SH):
                    p = qrows[e, pl.ds(h * HD, HD)] \
                        * kvrows[e, pl.ds(h * HD, HD)] * attr_v
                    sh = _splat16(plsc.cumsum(p), 15)
                    atn = jnp.where(lane_iota == h, sh, atn)
                w = jnp.exp(atn * 0.25)
                orows[e, pl.ds(128, 16)] = w
                for h in range(SH):
                    wh = _splat16(w, h)
                    orows[e, pl.ds(h * HD, HD)] = \
                        wh * kvrows[e, pl.ds(128 + h * HD, HD)]
                return 0

            lax.fori_loop(0, EC, ebody, 0)
            pltpu.sync_copy(orows, acc_sh.at[didx], add=True)
            return 0

        def guarded(ci, _):
            @pl.when(wid + ci * NW < NCHUNK)
            def _():
                chunk(ci, 0)
            return 0

        lax.fori_loop(0, KMAX, guarded, 0)
        _acc_writeback(acc_sh, out_hbm, cid, sid)

    return k(s2, d2, q_table, kv_table, attr, zeros).reshape(2, NP, 144)


FP = 102400
FC = 128
FW = FP // NW
FNCH = FW // FC


def _fgather_sc(root3, idx_pad):
    mesh = plsc.VectorSubcoreMesh(**_MESH)

    @functools.partial(
        pl.kernel,
        out_type=jax.ShapeDtypeStruct((FP, ENC), jnp.float32),
        mesh=mesh,
        compiler_params=pltpu.CompilerParams(use_tc_tiling_on_sc=False, needs_layout_passes=False),
        scratch_types=[
            pltpu.VMEM((FC,), jnp.int32),
            pltpu.VMEM((FC, ENC), jnp.float32),
        ],
    )
    def k(t_hbm, i_hbm, out_hbm, iv, rv):
        cid = lax.axis_index("c")
        sid = lax.axis_index("s")
        wid = sid * NC + cid
        fbase = wid * FW

        def chunk(ci, _):
            b = fbase + ci * FC
            pltpu.sync_copy(i_hbm.at[pl.ds(b, FC)], iv)
            pltpu.sync_copy(t_hbm.at[iv], rv)
            pltpu.sync_copy(rv, out_hbm.at[pl.ds(b, FC)])
            return 0

        lax.fori_loop(0, FNCH, chunk, 0)

    return k(root3, idx_pad)


def kernel(root_features, feedback_features, feedback_index, fringe_features,
           root_to_fringe_index, root_edge_index, root_edge_attr, W_attn,
           W_ctx2x, W_x2g, W_qkv, W_gate, W_up, W_down, W_fgate, b_fgate,
           W_r2f, b_r2f, rms_node_w, rms_root_w, rms_ffn_w):
    p = jnp.kron(jnp.eye(CH, dtype=jnp.float32), jnp.ones((1, HD), jnp.float32))
    wqkv4 = W_qkv.reshape(ENC, SH, HD, 3)
    wq = wqkv4[..., 0].reshape(ENC, ENC)
    wkv = jnp.concatenate(
        [wqkv4[..., 1].reshape(ENC, ENC), wqkv4[..., 2].reshape(ENC, ENC)],
        axis=1)
    hff = W_gate.shape[1]
    wg = jnp.pad(W_gate, ((0, 0), (0, HFFP - hff)))
    wu = jnp.pad(W_up, ((0, 0), (0, HFFP - hff)))
    wd = jnp.pad(W_down, ((0, HFFP - hff), (0, 0)))

    src = feedback_index[0].astype(jnp.int32)
    dst = feedback_index[1].astype(jnp.int32)
    s2 = root_edge_index[0].astype(jnp.int32)
    d2 = root_edge_index[1].astype(jnp.int32)

    zeros = jnp.zeros((NP, 144), jnp.float32)
    idx_pad = jnp.pad(root_to_fringe_index.astype(jnp.int32), (0, FP - F))

    src_table, xp_table = _pre1(feedback_features, root_features, W_attn,
                                W_ctx2x)
    acc1 = _edge1_sc(src, dst, src_table, xp_table, zeros)
    root1, q_table, kv_table = _mid(acc1, root_features, W_x2g, rms_node_w,
                                    wq, wkv, p)
    acc2 = _edge2_sc(s2, d2, q_table, kv_table, root_edge_attr, zeros)
    root3 = _post(acc2, root1, rms_root_w, wg, wu, wd, rms_ffn_w, p)
    gathered = _fgather_sc(root3, idx_pad)[:F]
    fringe_out = _fringe(fringe_features, gathered, W_fgate, b_fgate, W_r2f,
                         b_r2f)
    return (root3, fringe_out)
